# Initial kernel scaffold; baseline (speedup 1.0000x reference)
import jax
import jax.numpy as jnp
from jax.experimental import pallas as pl


def kernel(inputs):
    raise NotImplementedError("write your pallas kernel here")


# speedup vs baseline: 1.3666x; 1.3666x over previous
import jax
import jax.numpy as jnp
from jax import lax
from jax.experimental import pallas as pl
from jax.experimental.pallas import tpu as pltpu
from jax.experimental.pallas import tpu_sc as plsc

K = 8
L = 16
ROWS = 128
COLS = 32768
CHUNKS = COLS // L
NC, NS = 2, 16
NW = NC * NS
ROWS_PER_W = ROWS // NW
UNROLL = 4

_NEG_INF = float("-inf")


_GATHER_DNUMS = lax.GatherDimensionNumbers(
    offset_dims=(), collapsed_slice_dims=(0,), start_index_map=(0,))


def _lane_shuffle(v, s):
    idx = jnp.bitwise_xor(jnp.arange(L, dtype=jnp.int32), jnp.int32(s))
    return lax.gather(v, idx[:, None], _GATHER_DNUMS, slice_sizes=(1,),
                      mode=lax.GatherScatterMode.PROMISE_IN_BOUNDS)


def _all_lanes_max(v):
    for s in (1, 2, 4, 8):
        v = jnp.maximum(v, _lane_shuffle(v, s))
    return v


def _all_lanes_sum(v):
    for s in (1, 2, 4, 8):
        v = v + _lane_shuffle(v, s)
    return v


def _row_pivot(row_v):
    def scan_body(i, t):
        for u in range(UNROLL):
            v = row_v[pl.ds((i * UNROLL + u) * L, L)]
            tt = list(t)
            for s in range(K):
                hi = jnp.maximum(tt[s], v)
                v = jnp.minimum(tt[s], v)
                tt[s] = hi
            t = tuple(tt)
        return t

    t0 = tuple(jnp.full((L,), _NEG_INF, jnp.float32) for _ in range(K))
    t = lax.fori_loop(0, CHUNKS // UNROLL, scan_body, t0)

    def merge_body(_, carry):
        thr, pivot, done = carry
        m = jnp.full((L,), _NEG_INF, jnp.float32)
        for s in range(K):
            m = jnp.maximum(m, jnp.where(t[s] < thr, t[s], _NEG_INF))
        mm = _all_lanes_max(m)
        cnt = jnp.zeros((L,), jnp.float32)
        for s in range(K):
            cnt = cnt + jnp.where(t[s] >= mm, jnp.float32(1.0),
                                  jnp.float32(0.0))
        cnt = _all_lanes_sum(cnt)
        pivot = jnp.where(done > jnp.float32(0.5), pivot, mm)
        done = jnp.maximum(done, jnp.where(cnt >= jnp.float32(K),
                                           jnp.float32(1.0), jnp.float32(0.0)))
        return mm, pivot, done

    init = (jnp.full((L,), float("inf"), jnp.float32),
            jnp.full((L,), _NEG_INF, jnp.float32),
            jnp.zeros((L,), jnp.float32))
    _, pivot, _ = lax.fori_loop(0, K, merge_body, init)
    return pivot


def _body(x_hbm, out_hbm, row_v):
    cid = lax.axis_index("c")
    sid = lax.axis_index("s")
    wid = sid * NC + cid
    for k in range(ROWS_PER_W):
        r = wid * ROWS_PER_W + k
        pltpu.sync_copy(x_hbm.at[pl.ds(r * COLS, COLS)], row_v)
        pivot = _row_pivot(row_v)

        def thr_body(i, _):
            for u in range(UNROLL):
                j = (i * UNROLL + u) * L
                v = row_v[pl.ds(j, L)]
                row_v[pl.ds(j, L)] = jnp.where(v >= pivot, jnp.float32(1.0),
                                               jnp.float32(0.0))
            return 0

        lax.fori_loop(0, CHUNKS // UNROLL, thr_body, 0)
        pltpu.sync_copy(row_v, out_hbm.at[pl.ds(r * COLS, COLS)])


def kernel(inputs):
    mesh = plsc.VectorSubcoreMesh(core_axis_name="c", subcore_axis_name="s")
    out = pl.kernel(
        _body,
        out_type=jax.ShapeDtypeStruct((ROWS * COLS,), jnp.float32),
        mesh=mesh,
        scratch_types=[pltpu.VMEM((COLS,), jnp.float32)],
    )(inputs.reshape(ROWS * COLS))
    return out.reshape(ROWS, COLS)


# speedup vs baseline: 1.4647x; 1.0718x over previous
import jax
import jax.numpy as jnp
from jax import lax
from jax.experimental import pallas as pl
from jax.experimental.pallas import tpu as pltpu
from jax.experimental.pallas import tpu_sc as plsc

K = 8
L = 16
ROWS = 128
COLS = 32768
CHUNKS = COLS // L
NC, NS = 2, 16
NW = NC * NS
ROWS_PER_W = ROWS // NW
SEG = 4
UNROLL = 8

_NEG_INF = float("-inf")

_GATHER_DNUMS = lax.GatherDimensionNumbers(
    offset_dims=(), collapsed_slice_dims=(0,), start_index_map=(0,))


def _lane_shuffle(v, s):
    idx = jnp.bitwise_xor(jnp.arange(L, dtype=jnp.int32), jnp.int32(s))
    return lax.gather(v, idx[:, None], _GATHER_DNUMS, slice_sizes=(1,),
                      mode=lax.GatherScatterMode.PROMISE_IN_BOUNDS)


def _all_lanes_max(v):
    for s in (1, 2, 4, 8):
        v = jnp.maximum(v, _lane_shuffle(v, s))
    return v


def _all_lanes_sum(v):
    for s in (1, 2, 4, 8):
        v = v + _lane_shuffle(v, s)
    return v


def _row_pivot(row_v):
    def scan_body(i, t):
        tl = [list(seg) for seg in t]
        for u in range(SEG):
            v = row_v[pl.ds((i * SEG + u) * L, L)]
            for s in range(K):
                hi = jnp.maximum(tl[u][s], v)
                v = jnp.minimum(tl[u][s], v)
                tl[u][s] = hi
        return tuple(tuple(seg) for seg in tl)

    t0 = tuple(tuple(jnp.full((L,), _NEG_INF, jnp.float32) for _ in range(K))
               for _ in range(SEG))
    t = lax.fori_loop(0, CHUNKS // SEG, scan_body, t0)
    cand = [t[u][s] for u in range(SEG) for s in range(K)]

    def merge_body(_, carry):
        thr, pivot, done = carry
        m = jnp.full((L,), _NEG_INF, jnp.float32)
        for c in cand:
            m = jnp.maximum(m, jnp.where(c < thr, c, _NEG_INF))
        mm = _all_lanes_max(m)
        cnt = jnp.zeros((L,), jnp.float32)
        for c in cand:
            cnt = cnt + jnp.where(c >= mm, jnp.float32(1.0), jnp.float32(0.0))
        cnt = _all_lanes_sum(cnt)
        pivot = jnp.where(done > jnp.float32(0.5), pivot, mm)
        done = jnp.maximum(done, jnp.where(cnt >= jnp.float32(K),
                                           jnp.float32(1.0), jnp.float32(0.0)))
        return mm, pivot, done

    init = (jnp.full((L,), float("inf"), jnp.float32),
            jnp.full((L,), _NEG_INF, jnp.float32),
            jnp.zeros((L,), jnp.float32))
    _, pivot, _ = lax.fori_loop(0, K, merge_body, init)
    return pivot


def _body(x_hbm, out_hbm, buf0, buf1, in_sem):
    cid = lax.axis_index("c")
    sid = lax.axis_index("s")
    wid = sid * NC + cid
    bufs = (buf0, buf1)

    def row_off(k):
        return pl.ds((wid * ROWS_PER_W + k) * COLS, COLS)

    pending = pltpu.async_copy(x_hbm.at[row_off(0)], bufs[0], in_sem)
    for k in range(ROWS_PER_W):
        if k + 1 < ROWS_PER_W:
            nxt = pltpu.async_copy(x_hbm.at[row_off(k + 1)],
                                   bufs[(k + 1) % 2], in_sem)
        pending.wait()
        buf = bufs[k % 2]
        pivot = _row_pivot(buf)

        def thr_body(i, _):
            for u in range(UNROLL):
                j = (i * UNROLL + u) * L
                v = buf[pl.ds(j, L)]
                buf[pl.ds(j, L)] = jnp.where(v >= pivot, jnp.float32(1.0),
                                             jnp.float32(0.0))
            return 0

        lax.fori_loop(0, CHUNKS // UNROLL, thr_body, 0)
        pltpu.sync_copy(buf, out_hbm.at[row_off(k)])
        if k + 1 < ROWS_PER_W:
            pending = nxt


def kernel(inputs):
    mesh = plsc.VectorSubcoreMesh(core_axis_name="c", subcore_axis_name="s")
    out = pl.kernel(
        _body,
        out_type=jax.ShapeDtypeStruct((ROWS * COLS,), jnp.float32),
        mesh=mesh,
        scratch_types=[pltpu.VMEM((COLS,), jnp.float32),
                       pltpu.VMEM((COLS,), jnp.float32),
                       pltpu.SemaphoreType.DMA],
    )(inputs.reshape(ROWS * COLS))
    return out.reshape(ROWS, COLS)


# speedup vs baseline: 1.7787x; 1.2144x over previous
import jax
import jax.numpy as jnp
from jax import lax
from jax.experimental import pallas as pl
from jax.experimental.pallas import tpu as pltpu
from jax.experimental.pallas import tpu_sc as plsc

K = 8
L = 16
ROWS = 128
COLS = 32768
CHUNKS = COLS // L
GSIZE = 8
NGRP = CHUNKS // GSIZE
SGSIZE = 8
NSG = NGRP // SGSIZE
NC, NS = 2, 16
NW = NC * NS
ROWS_PER_W = ROWS // NW
NBUF = 3
UNROLL = 8

_NEG_INF = float("-inf")

_GATHER_DNUMS = lax.GatherDimensionNumbers(
    offset_dims=(), collapsed_slice_dims=(0,), start_index_map=(0,))


def _lane_shuffle(v, s):
    idx = jnp.bitwise_xor(jnp.arange(L, dtype=jnp.int32), jnp.int32(s))
    return lax.gather(v, idx[:, None], _GATHER_DNUMS, slice_sizes=(1,),
                      mode=lax.GatherScatterMode.PROMISE_IN_BOUNDS)


def _all_lanes_max(v):
    for s in (1, 2, 4, 8):
        v = jnp.maximum(v, _lane_shuffle(v, s))
    return v


def _all_lanes_sum(v):
    for s in (1, 2, 4, 8):
        v = v + _lane_shuffle(v, s)
    return v


def _max_tree(vs):
    while len(vs) > 1:
        vs = [jnp.maximum(vs[i], vs[i + 1]) for i in range(0, len(vs) - 1, 2)] \
            + ([vs[-1]] if len(vs) % 2 else [])
    return vs[0]


def _insert8(t, v):
    tt = list(t)
    for s in range(K):
        hi = jnp.maximum(tt[s], v)
        v = jnp.minimum(tt[s], v)
        tt[s] = hi
    return tuple(tt)


def _row_pivot(row_v, gmax_v, sgmax_v, tacc_v):
    def grp_body(g, _):
        base = g * GSIZE * L
        vs = [row_v[pl.ds(base + c * L, L)] for c in range(GSIZE)]
        gmax_v[pl.ds(g * L, L)] = _max_tree(vs)
        return 0

    lax.fori_loop(0, NGRP, grp_body, 0)

    def sg_body(s, _):
        base = s * SGSIZE * L
        vs = [gmax_v[pl.ds(base + c * L, L)] for c in range(SGSIZE)]
        sgmax_v[pl.ds(s * L, L)] = _max_tree(vs)
        return 0

    lax.fori_loop(0, NSG, sg_body, 0)

    def est_body(s, t):
        return _insert8(t, sgmax_v[pl.ds(s * L, L)])

    t0 = tuple(jnp.full((L,), _NEG_INF, jnp.float32) for _ in range(K))
    t_est_lanes = lax.fori_loop(0, NSG, est_body, t0)
    t_est = _all_lanes_max(t_est_lanes[K - 1])

    t_est_s = t_est[0]
    for s in range(K):
        tacc_v[pl.ds(s * L, L)] = jnp.full((L,), _NEG_INF, jnp.float32)

    def sg_scan(si, _):
        sm = _all_lanes_max(sgmax_v[pl.ds(si * L, L)])

        @pl.when(sm[0] >= t_est_s)
        def _proc_sg():
            def g_scan(gj, _):
                g = si * SGSIZE + gj
                gm = _all_lanes_max(gmax_v[pl.ds(g * L, L)])

                @pl.when(gm[0] >= t_est_s)
                def _proc_g():
                    t = tuple(tacc_v[pl.ds(s * L, L)] for s in range(K))
                    base = g * GSIZE * L
                    for c in range(GSIZE):
                        t = _insert8(t, row_v[pl.ds(base + c * L, L)])
                    for s in range(K):
                        tacc_v[pl.ds(s * L, L)] = t[s]

                return 0

            lax.fori_loop(0, SGSIZE, g_scan, 0)

        return 0

    lax.fori_loop(0, NSG, sg_scan, 0)
    t = tuple(tacc_v[pl.ds(s * L, L)] for s in range(K))

    def merge_body(_, carry):
        thr, pivot, done = carry
        m = jnp.full((L,), _NEG_INF, jnp.float32)
        for c in t:
            m = jnp.maximum(m, jnp.where(c < thr, c, _NEG_INF))
        mm = _all_lanes_max(m)
        cnt = jnp.zeros((L,), jnp.float32)
        for c in t:
            cnt = cnt + jnp.where(c >= mm, jnp.float32(1.0), jnp.float32(0.0))
        cnt = _all_lanes_sum(cnt)
        pivot = jnp.where(done > jnp.float32(0.5), pivot, mm)
        done = jnp.maximum(done, jnp.where(cnt >= jnp.float32(K),
                                           jnp.float32(1.0), jnp.float32(0.0)))
        return mm, pivot, done

    init = (jnp.full((L,), float("inf"), jnp.float32),
            jnp.full((L,), _NEG_INF, jnp.float32),
            jnp.zeros((L,), jnp.float32))
    _, pivot, _ = lax.fori_loop(0, K, merge_body, init)
    return pivot


def _body(x_hbm, out_hbm, b0, b1, b2, gmax_v, sgmax_v, tacc_v, in_sem, out_sem):
    cid = lax.axis_index("c")
    sid = lax.axis_index("s")
    wid = sid * NC + cid
    bufs = (b0, b1, b2)

    def row(k):
        return wid * ROWS_PER_W + k

    in_h = [pltpu.async_copy(x_hbm.at[row(0)], bufs[0], in_sem)]
    out_h = []
    for k in range(ROWS_PER_W):
        if k + 1 < ROWS_PER_W:
            if k - 2 >= 0:
                out_h[k - 2].wait()
            in_h.append(pltpu.async_copy(x_hbm.at[row(k + 1)],
                                         bufs[(k + 1) % NBUF], in_sem))
        in_h[k].wait()
        buf = bufs[k % NBUF]
        pivot = _row_pivot(buf, gmax_v, sgmax_v, tacc_v)

        def thr_body(i, _):
            for u in range(UNROLL):
                j = (i * UNROLL + u) * L
                v = buf[pl.ds(j, L)]
                buf[pl.ds(j, L)] = jnp.where(v >= pivot, jnp.float32(1.0),
                                             jnp.float32(0.0))
            return 0

        lax.fori_loop(0, CHUNKS // UNROLL, thr_body, 0)
        out_h.append(pltpu.async_copy(buf, out_hbm.at[row(k)], out_sem))
    for k in range(max(0, ROWS_PER_W - 3), ROWS_PER_W):
        out_h[k].wait()


def kernel(inputs):
    mesh = plsc.VectorSubcoreMesh(core_axis_name="c", subcore_axis_name="s")
    return pl.kernel(
        _body,
        out_type=jax.ShapeDtypeStruct((ROWS, COLS), jnp.float32),
        mesh=mesh,
        scratch_types=[pltpu.VMEM((COLS,), jnp.float32),
                       pltpu.VMEM((COLS,), jnp.float32),
                       pltpu.VMEM((COLS,), jnp.float32),
                       pltpu.VMEM((NGRP * L,), jnp.float32),
                       pltpu.VMEM((NSG * L,), jnp.float32),
                       pltpu.VMEM((K * L,), jnp.float32),
                       pltpu.SemaphoreType.DMA,
                       pltpu.SemaphoreType.DMA],
    )(inputs)


# speedup vs baseline: 2.5345x; 1.4249x over previous
import jax
import jax.numpy as jnp
from jax import lax
from jax.experimental import pallas as pl
from jax.experimental.pallas import tpu as pltpu
from jax.experimental.pallas import tpu_sc as plsc

K = 8
L = 16
ROWS = 128
COLS = 32768
CHUNKS = COLS // L
GSIZE = 8
NGRP = CHUNKS // GSIZE
NACC = 4
NC, NS = 2, 16
NW = NC * NS
ROWS_PER_W = ROWS // NW
NBUF = 3
UNROLL = 8

_NEG_INF = float("-inf")

_GATHER_DNUMS = lax.GatherDimensionNumbers(
    offset_dims=(), collapsed_slice_dims=(0,), start_index_map=(0,))


def _lane_shuffle(v, s):
    idx = jnp.bitwise_xor(jnp.arange(L, dtype=jnp.int32), jnp.int32(s))
    return lax.gather(v, idx[:, None], _GATHER_DNUMS, slice_sizes=(1,),
                      mode=lax.GatherScatterMode.PROMISE_IN_BOUNDS)


def _all_lanes_max(v):
    for s in (1, 2, 4, 8):
        v = jnp.maximum(v, _lane_shuffle(v, s))
    return v


def _all_lanes_sum(v):
    for s in (1, 2, 4, 8):
        v = v + _lane_shuffle(v, s)
    return v


def _max_tree(vs):
    while len(vs) > 1:
        vs = [jnp.maximum(vs[i], vs[i + 1]) for i in range(0, len(vs) - 1, 2)] \
            + ([vs[-1]] if len(vs) % 2 else [])
    return vs[0]


def _insert8(t, v):
    tt = list(t)
    for s in range(K):
        hi = jnp.maximum(tt[s], v)
        v = jnp.minimum(tt[s], v)
        tt[s] = hi
    return tuple(tt)


def _neg_tuple():
    return tuple(jnp.full((L,), _NEG_INF, jnp.float32) for _ in range(K))


def _merge_pivot(cand):
    def merge_body(_, carry):
        thr, pivot, done = carry
        m = jnp.full((L,), _NEG_INF, jnp.float32)
        for c in cand:
            m = jnp.maximum(m, jnp.where(c < thr, c, _NEG_INF))
        mm = _all_lanes_max(m)
        cnt = jnp.zeros((L,), jnp.float32)
        for c in cand:
            cnt = cnt + jnp.where(c >= mm, jnp.float32(1.0), jnp.float32(0.0))
        cnt = _all_lanes_sum(cnt)
        pivot = jnp.where(done > jnp.float32(0.5), pivot, mm)
        done = jnp.maximum(done, jnp.where(cnt >= jnp.float32(K),
                                           jnp.float32(1.0), jnp.float32(0.0)))
        return mm, pivot, done

    init = (jnp.full((L,), float("inf"), jnp.float32),
            jnp.full((L,), _NEG_INF, jnp.float32),
            jnp.zeros((L,), jnp.float32))
    _, pivot, _ = lax.fori_loop(0, K, merge_body, init)
    return pivot


def _row_pivot(row_v, piv_v):
    def grp_body(i, accs):
        out = []
        for u in range(NACC):
            base = (i * NACC + u) * GSIZE * L
            vs = [row_v[pl.ds(base + c * L, L)] for c in range(GSIZE)]
            out.append(_insert8(accs[u], _max_tree(vs)))
        return tuple(out)

    accs = lax.fori_loop(0, NGRP // NACC, grp_body,
                         tuple(_neg_tuple() for _ in range(NACC)))
    t = accs[0]
    for u in range(1, NACC):
        for s in range(K):
            t = _insert8(t, accs[u][s])
    t_est = _merge_pivot(t)

    def cnt_body(i, c):
        for u in range(UNROLL):
            v = row_v[pl.ds((i * UNROLL + u) * L, L)]
            c = c + jnp.where(v >= t_est, jnp.float32(1.0), jnp.float32(0.0))
        return c

    c = lax.fori_loop(0, CHUNKS // UNROLL, cnt_body,
                      jnp.zeros((L,), jnp.float32))
    cnt = _all_lanes_sum(c)[0]
    piv_v[pl.ds(0, L)] = t_est

    @pl.when(cnt != jnp.float32(K))
    def _fallback():
        def scan_body(i, accs):
            out = []
            for u in range(NACC):
                v = row_v[pl.ds((i * NACC + u) * L, L)]
                out.append(_insert8(accs[u], v))
            return tuple(out)

        faccs = lax.fori_loop(0, CHUNKS // NACC, scan_body,
                              tuple(_neg_tuple() for _ in range(NACC)))
        cand = [faccs[u][s] for u in range(NACC) for s in range(K)]
        piv_v[pl.ds(0, L)] = _merge_pivot(cand)


def _body(x_hbm, out_hbm, b0, b1, b2, piv_v, in_sem, out_sem):
    cid = lax.axis_index("c")
    sid = lax.axis_index("s")
    wid = sid * NC + cid
    bufs = (b0, b1, b2)

    def row(k):
        return wid * ROWS_PER_W + k

    in_h = [pltpu.async_copy(x_hbm.at[row(0)], bufs[0], in_sem)]
    out_h = []
    for k in range(ROWS_PER_W):
        if k + 1 < ROWS_PER_W:
            if k - 2 >= 0:
                out_h[k - 2].wait()
            in_h.append(pltpu.async_copy(x_hbm.at[row(k + 1)],
                                         bufs[(k + 1) % NBUF], in_sem))
        in_h[k].wait()
        buf = bufs[k % NBUF]
        _row_pivot(buf, piv_v)
        pivot = piv_v[pl.ds(0, L)]

        def thr_body(i, _):
            for u in range(UNROLL):
                j = (i * UNROLL + u) * L
                v = buf[pl.ds(j, L)]
                buf[pl.ds(j, L)] = jnp.where(v >= pivot, jnp.float32(1.0),
                                             jnp.float32(0.0))
            return 0

        lax.fori_loop(0, CHUNKS // UNROLL, thr_body, 0)
        out_h.append(pltpu.async_copy(buf, out_hbm.at[row(k)], out_sem))
    for k in range(max(0, ROWS_PER_W - 3), ROWS_PER_W):
        out_h[k].wait()


def kernel(inputs):
    mesh = plsc.VectorSubcoreMesh(core_axis_name="c", subcore_axis_name="s")
    return pl.kernel(
        _body,
        out_type=jax.ShapeDtypeStruct((ROWS, COLS), jnp.float32),
        mesh=mesh,
        scratch_types=[pltpu.VMEM((COLS,), jnp.float32),
                       pltpu.VMEM((COLS,), jnp.float32),
                       pltpu.VMEM((COLS,), jnp.float32),
                       pltpu.VMEM((L,), jnp.float32),
                       pltpu.SemaphoreType.DMA,
                       pltpu.SemaphoreType.DMA],
    )(inputs)


# speedup vs baseline: 2.6921x; 1.0621x over previous
import jax
import jax.numpy as jnp
from jax import lax
from jax.experimental import pallas as pl
from jax.experimental.pallas import tpu as pltpu
from jax.experimental.pallas import tpu_sc as plsc

K = 8
L = 16
ROWS = 128
COLS = 32768
CHUNKS = COLS // L
GSIZE = 16
NGRP = CHUNKS // GSIZE
NACC = 4
NC, NS = 2, 16
NW = NC * NS
ROWS_PER_W = ROWS // NW
NBUF = 3
UNROLL = 16

_NEG_INF = float("-inf")

_GATHER_DNUMS = lax.GatherDimensionNumbers(
    offset_dims=(), collapsed_slice_dims=(0,), start_index_map=(0,))


def _lane_shuffle(v, s):
    idx = jnp.bitwise_xor(jnp.arange(L, dtype=jnp.int32), jnp.int32(s))
    return lax.gather(v, idx[:, None], _GATHER_DNUMS, slice_sizes=(1,),
                      mode=lax.GatherScatterMode.PROMISE_IN_BOUNDS)


def _all_lanes_max(v):
    for s in (1, 2, 4, 8):
        v = jnp.maximum(v, _lane_shuffle(v, s))
    return v


def _all_lanes_sum(v):
    for s in (1, 2, 4, 8):
        v = v + _lane_shuffle(v, s)
    return v


def _max_tree(vs):
    while len(vs) > 1:
        vs = [jnp.maximum(vs[i], vs[i + 1]) for i in range(0, len(vs) - 1, 2)] \
            + ([vs[-1]] if len(vs) % 2 else [])
    return vs[0]


def _insert8(t, v):
    tt = list(t)
    for s in range(K):
        hi = jnp.maximum(tt[s], v)
        v = jnp.minimum(tt[s], v)
        tt[s] = hi
    return tuple(tt)


def _neg_tuple():
    return tuple(jnp.full((L,), _NEG_INF, jnp.float32) for _ in range(K))


def _merge_pivot(cand):
    def merge_body(_, carry):
        thr, pivot, done = carry
        m = jnp.full((L,), _NEG_INF, jnp.float32)
        for c in cand:
            m = jnp.maximum(m, jnp.where(c < thr, c, _NEG_INF))
        mm = _all_lanes_max(m)
        cnt = jnp.zeros((L,), jnp.float32)
        for c in cand:
            cnt = cnt + jnp.where(c >= mm, jnp.float32(1.0), jnp.float32(0.0))
        cnt = _all_lanes_sum(cnt)
        pivot = jnp.where(done > jnp.float32(0.5), pivot, mm)
        done = jnp.maximum(done, jnp.where(cnt >= jnp.float32(K),
                                           jnp.float32(1.0), jnp.float32(0.0)))
        return mm, pivot, done

    init = (jnp.full((L,), float("inf"), jnp.float32),
            jnp.full((L,), _NEG_INF, jnp.float32),
            jnp.zeros((L,), jnp.float32))
    _, pivot, _ = lax.fori_loop(0, K, merge_body, init)
    return pivot


def _row_pivot(row_v, piv_v):
    def grp_body(i, accs):
        out = []
        for u in range(NACC):
            base = (i * NACC + u) * GSIZE * L
            vs = [row_v[pl.ds(base + c * L, L)] for c in range(GSIZE)]
            out.append(_insert8(accs[u], _max_tree(vs)))
        return tuple(out)

    accs = lax.fori_loop(0, NGRP // NACC, grp_body,
                         tuple(_neg_tuple() for _ in range(NACC)))
    t = accs[0]
    for u in range(1, NACC):
        for s in range(K):
            t = _insert8(t, accs[u][s])
    t_est = _merge_pivot(t)

    def cnt_body(i, cs):
        cs = list(cs)
        for u in range(UNROLL):
            v = row_v[pl.ds((i * UNROLL + u) * L, L)]
            cs[u % 4] = cs[u % 4] + jnp.where(v >= t_est, jnp.float32(1.0),
                                              jnp.float32(0.0))
        return tuple(cs)

    cs = lax.fori_loop(0, CHUNKS // UNROLL, cnt_body,
                       tuple(jnp.zeros((L,), jnp.float32) for _ in range(4)))
    cnt = _all_lanes_sum(cs[0] + cs[1] + cs[2] + cs[3])[0]
    piv_v[pl.ds(0, L)] = t_est

    @pl.when(cnt != jnp.float32(K))
    def _fallback():
        def scan_body(i, accs):
            out = []
            for u in range(NACC):
                v = row_v[pl.ds((i * NACC + u) * L, L)]
                out.append(_insert8(accs[u], v))
            return tuple(out)

        faccs = lax.fori_loop(0, CHUNKS // NACC, scan_body,
                              tuple(_neg_tuple() for _ in range(NACC)))
        cand = [faccs[u][s] for u in range(NACC) for s in range(K)]
        piv_v[pl.ds(0, L)] = _merge_pivot(cand)


def _body(x_hbm, out_hbm, b0, b1, b2, piv_v, in_sem, out_sem):
    cid = lax.axis_index("c")
    sid = lax.axis_index("s")
    wid = sid * NC + cid
    bufs = (b0, b1, b2)

    def row(k):
        return wid * ROWS_PER_W + k

    in_h = [pltpu.async_copy(x_hbm.at[row(0)], bufs[0], in_sem)]
    out_h = []
    for k in range(ROWS_PER_W):
        if k + 1 < ROWS_PER_W:
            if k - 2 >= 0:
                out_h[k - 2].wait()
            in_h.append(pltpu.async_copy(x_hbm.at[row(k + 1)],
                                         bufs[(k + 1) % NBUF], in_sem))
        in_h[k].wait()
        buf = bufs[k % NBUF]
        _row_pivot(buf, piv_v)
        pivot = piv_v[pl.ds(0, L)]

        def thr_body(i, _):
            for u in range(UNROLL):
                j = (i * UNROLL + u) * L
                v = buf[pl.ds(j, L)]
                buf[pl.ds(j, L)] = jnp.where(v >= pivot, jnp.float32(1.0),
                                             jnp.float32(0.0))
            return 0

        lax.fori_loop(0, CHUNKS // UNROLL, thr_body, 0)
        out_h.append(pltpu.async_copy(buf, out_hbm.at[row(k)], out_sem))
    for k in range(max(0, ROWS_PER_W - 3), ROWS_PER_W):
        out_h[k].wait()


def kernel(inputs):
    mesh = plsc.VectorSubcoreMesh(core_axis_name="c", subcore_axis_name="s")
    return pl.kernel(
        _body,
        out_type=jax.ShapeDtypeStruct((ROWS, COLS), jnp.float32),
        mesh=mesh,
        scratch_types=[pltpu.VMEM((COLS,), jnp.float32),
                       pltpu.VMEM((COLS,), jnp.float32),
                       pltpu.VMEM((COLS,), jnp.float32),
                       pltpu.VMEM((L,), jnp.float32),
                       pltpu.SemaphoreType.DMA,
                       pltpu.SemaphoreType.DMA],
    )(inputs)
